# baseline (speedup 1.0000x reference)
import functools

import jax
import jax.numpy as jnp
from jax import lax
from jax.experimental import pallas as pl
from jax.experimental.pallas import tpu as pltpu
from jax.experimental.pallas import tpu_sc as plsc

N_NODES = 100000
N_EDGES = 1600000
HIDDEN = 32
HALF = 16
SUB = 128
CH = 5
CHUNK = CH * SUB
N_CHUNKS = N_EDGES // CHUNK
N_TILES = 16
ROWS_PER_TILE = N_NODES // N_TILES
ZROWS = 250

_sc_mesh = plsc.VectorSubcoreMesh(core_axis_name="c", subcore_axis_name="s")


@functools.partial(
    pl.kernel,
    out_type=jax.ShapeDtypeStruct((2, N_NODES, HALF), jnp.float32),
    mesh=_sc_mesh,
    scratch_types=[
        pltpu.VMEM_SHARED((N_NODES, HALF), jnp.float32),
        pltpu.VMEM((CH, SUB), jnp.int32),
        pltpu.VMEM((CH, SUB), jnp.int32),
        pltpu.VMEM((CHUNK, HALF), jnp.float32),
        pltpu.VMEM((CHUNK, HALF), jnp.float32),
        pltpu.VMEM((ZROWS, HALF), jnp.float32),
        pltpu.SemaphoreType.DMA,
        pltpu.SemaphoreType.DMA,
        pltpu.SemaphoreType.DMA,
        pltpu.SemaphoreType.DMA,
    ],
    compiler_params=pltpu.CompilerParams(use_tc_tiling_on_sc=False),
)
def _sc_scatter(col_hbm, ea_hbm, out_hbm, acc, colv0, colv1, eav0, eav1, zb,
                sc0, sc1, se0, se1):
    c = lax.axis_index("c")
    s = lax.axis_index("s")
    c16 = c * HALF

    def _load(j, colv, eav, semc, seme):
        for k in range(CH):
            pltpu.async_copy(
                col_hbm.at[pl.ds(j * CHUNK + k * SUB, SUB)], colv.at[k], semc
            )
        pltpu.async_copy(
            ea_hbm.at[pl.ds(j * CHUNK, CHUNK), pl.ds(c16, HALF)], eav, seme
        )

    def _drain(j, colv, eav, semc, seme):
        for k in range(CH):
            pltpu.make_async_copy(
                col_hbm.at[pl.ds(j * CHUNK + k * SUB, SUB)], colv.at[k], semc
            ).wait()
        pltpu.make_async_copy(
            ea_hbm.at[pl.ds(j * CHUNK, CHUNK), pl.ds(c16, HALF)], eav, seme
        ).wait()

    def _scatter(colv, eav):
        for k in range(CH):
            pltpu.sync_copy(
                eav.at[pl.ds(k * SUB, SUB)], acc.at[colv.at[k]], add=True
            )

    lo = s * N_CHUNKS // N_TILES
    hi = (s + 1) * N_CHUNKS // N_TILES
    n = hi - lo

    _load(lo, colv0, eav0, sc0, se0)

    @pl.when(n > 1)
    def _():
        _load(lo + 1, colv1, eav1, sc1, se1)

    def _zero_row(i, _):
        zb[i, :] = jnp.zeros((HALF,), jnp.float32)
        return _

    lax.fori_loop(0, ZROWS, _zero_row, None, unroll=4)
    row0 = s * ROWS_PER_TILE
    for k in range(ROWS_PER_TILE // ZROWS):
        pltpu.sync_copy(zb, acc.at[pl.ds(row0 + k * ZROWS, ZROWS)])
    plsc.subcore_barrier()

    def _pair(p, _):
        j0 = lo + 2 * p
        _drain(j0, colv0, eav0, sc0, se0)
        _scatter(colv0, eav0)

        @pl.when(j0 + 2 < hi)
        def _():
            _load(j0 + 2, colv0, eav0, sc0, se0)

        _drain(j0 + 1, colv1, eav1, sc1, se1)
        _scatter(colv1, eav1)

        @pl.when(j0 + 3 < hi)
        def _():
            _load(j0 + 3, colv1, eav1, sc1, se1)

        return _

    lax.fori_loop(0, n // 2, _pair, None)

    @pl.when(n % 2 == 1)
    def _():
        j = lo + (n // 2) * 2
        _drain(j, colv0, eav0, sc0, se0)
        _scatter(colv0, eav0)

    plsc.subcore_barrier()

    pltpu.sync_copy(
        acc.at[pl.ds(row0, ROWS_PER_TILE)],
        out_hbm.at[c, pl.ds(row0, ROWS_PER_TILE)],
    )


_MLP_R = 1000


def _mlp1_body(x_ref, w1a_ref, b1_ref, h_ref):
    h_ref[...] = (
        jnp.dot(x_ref[...], w1a_ref[...], preferred_element_type=jnp.float32)
        + b1_ref[...]
    )


def _mlp1(x, W1a, b1):
    return pl.pallas_call(
        _mlp1_body,
        grid=(N_NODES // _MLP_R,),
        in_specs=[
            pl.BlockSpec((_MLP_R, 128), lambda i: (i, 0)),
            pl.BlockSpec((128, 32), lambda i: (0, 0)),
            pl.BlockSpec((1, 32), lambda i: (0, 0)),
        ],
        out_specs=pl.BlockSpec((_MLP_R, 32), lambda i: (i, 0)),
        out_shape=jax.ShapeDtypeStruct((N_NODES, 32), jnp.float32),
    )(x, W1a, b1)


def _mlp2_body(h1_ref, agg_ref, w1b_ref, w2_ref, b2_ref, o_ref):
    a = jnp.concatenate([agg_ref[0], agg_ref[1]], axis=1)
    h = h1_ref[...] + jnp.dot(a, w1b_ref[...],
                              preferred_element_type=jnp.float32)
    h = jnp.maximum(h, 0.0)
    o = jnp.dot(h, w2_ref[...], preferred_element_type=jnp.float32) + b2_ref[...]
    o_ref[...] = jnp.maximum(o, 0.0)


def _mlp2(h1, agg2, W1b, W2, b2):
    return pl.pallas_call(
        _mlp2_body,
        grid=(N_NODES // _MLP_R,),
        in_specs=[
            pl.BlockSpec((_MLP_R, 32), lambda i: (i, 0)),
            pl.BlockSpec((2, _MLP_R, HALF), lambda i: (0, i, 0)),
            pl.BlockSpec((32, 32), lambda i: (0, 0)),
            pl.BlockSpec((32, 32), lambda i: (0, 0)),
            pl.BlockSpec((1, 32), lambda i: (0, 0)),
        ],
        out_specs=pl.BlockSpec((_MLP_R, 32), lambda i: (i, 0)),
        out_shape=jax.ShapeDtypeStruct((N_NODES, 32), jnp.float32),
    )(h1, agg2, W1b, W2, b2)


def kernel(x, edge_index, edge_attr, u, batch, W1, b1, W2, b2):
    col = edge_index[1].astype(jnp.int32)
    agg2 = _sc_scatter(col, edge_attr)
    h1 = _mlp1(x, W1[:128], b1.reshape(1, 32))
    return _mlp2(h1, agg2, W1[128:], W2, b2.reshape(1, 32))
